# Initial kernel scaffold; baseline (speedup 1.0000x reference)
import jax
import jax.numpy as jnp
from jax.experimental import pallas as pl


def kernel(local_embeddings, local_labels, weight, perm_rand):
    raise NotImplementedError("write your pallas kernel here")


# speedup vs baseline: 1.8326x; 1.8326x over previous
import functools

import jax
import jax.numpy as jnp
from jax.experimental import pallas as pl
from jax.experimental.pallas import tpu as pltpu

_NUM_CLASSES = 100000
_NUM_SAMPLE = 20000
_BATCH = 1024
_EMBED = 512
_S = 64.0
_COS_M = 0.8775825618903728
_SIN_M = 0.479425538604203
_NEG = -1e30


def _fused_ce_kernel(emb_ref, w_ref, lab_ref, out_ref,
                     nemb_ref, m_ref, s_ref, t_ref,
                     *, blk, n_valid, nsteps):
    j = pl.program_id(0)

    @pl.when(j == 0)
    def _init():
        e = emb_ref[...]
        inv = jax.lax.rsqrt(jnp.sum(e * e, axis=1, keepdims=True))
        nemb_ref[...] = e * inv
        m_ref[...] = jnp.full_like(m_ref, _NEG)
        s_ref[...] = jnp.zeros_like(s_ref)
        t_ref[...] = jnp.zeros_like(t_ref)

    w = w_ref[...]
    winv = jax.lax.rsqrt(jnp.maximum(jnp.sum(w * w, axis=1, keepdims=True),
                                     1e-30))
    nw = w * winv
    cos = jnp.dot(nemb_ref[...], nw.T, preferred_element_type=jnp.float32)
    cos = jnp.clip(cos, -1.0, 1.0)

    col = j * blk + jax.lax.broadcasted_iota(jnp.int32, cos.shape, 1)
    valid = col < n_valid
    is_t = lab_ref[...] == col
    tc = jnp.sum(jnp.where(is_t, cos, 0.0), axis=1, keepdims=True)
    has_t = jnp.sum(is_t.astype(jnp.float32), axis=1, keepdims=True) > 0.0
    tcc = jnp.clip(tc, -1.0 + 1e-7, 1.0 - 1e-7)
    marg = (tcc * _COS_M - jnp.sqrt(1.0 - tcc * tcc) * _SIN_M) * _S
    logits = jnp.where(is_t, marg, cos * _S)
    logits = jnp.where(valid, logits, _NEG)

    m_prev = m_ref[...]
    m_new = jnp.maximum(m_prev, jnp.max(logits, axis=1, keepdims=True))
    s_ref[...] = (s_ref[...] * jnp.exp(m_prev - m_new)
                  + jnp.sum(jnp.exp(logits - m_new), axis=1, keepdims=True))
    m_ref[...] = m_new
    t_ref[...] = t_ref[...] + jnp.where(has_t, marg, 0.0)

    @pl.when(j == nsteps - 1)
    def _fin():
        logprob = t_ref[...] - m_ref[...] - jnp.log(s_ref[...])
        out_ref[...] = -jnp.maximum(logprob, jnp.log(jnp.float32(1e-30)))


def _fused_ce(norm_src_emb, w_act_padded, labels_r, blk, n_valid,
              interpret=False):
    npad = w_act_padded.shape[0]
    nsteps = npad // blk
    out = pl.pallas_call(
        functools.partial(_fused_ce_kernel, blk=blk, n_valid=n_valid,
                          nsteps=nsteps),
        grid=(nsteps,),
        in_specs=[
            pl.BlockSpec((_BATCH, _EMBED), lambda j: (0, 0)),
            pl.BlockSpec((blk, _EMBED), lambda j: (j, 0)),
            pl.BlockSpec((_BATCH, 1), lambda j: (0, 0)),
        ],
        out_specs=pl.BlockSpec((_BATCH, 1), lambda j: (0, 0)),
        out_shape=jax.ShapeDtypeStruct((_BATCH, 1), jnp.float32),
        scratch_shapes=[
            pltpu.VMEM((_BATCH, _EMBED), jnp.float32),
            pltpu.VMEM((_BATCH, 1), jnp.float32),
            pltpu.VMEM((_BATCH, 1), jnp.float32),
            pltpu.VMEM((_BATCH, 1), jnp.float32),
        ],
        interpret=interpret,
    )(norm_src_emb, w_act_padded, labels_r)
    return out


def kernel(local_embeddings, local_labels, weight, perm_rand):
    labels = local_labels.astype(jnp.int32)
    perm = perm_rand.at[labels].set(2.0)
    _, index = jax.lax.top_k(perm, _NUM_SAMPLE)
    index = jnp.sort(index)
    labels_r = jnp.searchsorted(index, labels).astype(jnp.int32)
    w_act = jnp.take(weight, index, axis=0)

    npad = 20480
    w_pad = jnp.pad(w_act, ((0, npad - _NUM_SAMPLE), (0, 0)))
    per_row = _fused_ce(local_embeddings, w_pad,
                        labels_r.reshape(_BATCH, 1), blk=2048,
                        n_valid=_NUM_SAMPLE)
    return jnp.mean(per_row)
